# baseline (speedup 1.0000x reference)
import functools

import jax
import jax.numpy as jnp
from jax import lax
from jax.experimental import pallas as pl
from jax.experimental.pallas import tpu as pltpu
from jax.experimental.pallas import tpu_sc as plsc

K = 16
NSLAB = 16
LANES = 128
_HIGH = lax.Precision.HIGHEST
_FAST = lax.Precision.DEFAULT
_INTMAX = 0x7FFFFFFF
_KEEP = 4

_SC_CORES = 2
_SC_SUBCORES = 16
_SC_WORKERS = _SC_CORES * _SC_SUBCORES
_SC_CHUNK = 128


def _dot(a, b, precision=_HIGH):
    return lax.dot_general(a, b, (((1,), (0,)), ((), ())),
                           precision=precision,
                           preferred_element_type=jnp.float32)


def _bitonic_ce_pairs(n):
    pairs = []
    k = 2
    while k <= n:
        j = k // 2
        while j >= 1:
            for i in range(n):
                l = i ^ j
                if l > i:
                    pairs.append((i, l, (i & k) == 0))
            j //= 2
        k *= 2
    return pairs


def _pruned_bitonic(n, n_outputs):
    pairs = _bitonic_ce_pairs(n)
    needed = set(range(n_outputs))
    flags = []
    for a, b, asc in reversed(pairs):
        use = a in needed or b in needed
        flags.append(use)
        if use:
            needed.add(a)
            needed.add(b)
    flags.reverse()
    return [p for p, f in zip(pairs, flags) if f]


def _topk_body(ptsT_ref, pts_ref, ftile_ref, W0bT_ref, idx_ref, g_ref,
               *, n_pts, n_rows):
    P = n_pts
    R = n_rows
    nslab = P // LANES
    keep = min(_KEEP, nslab)
    n = pl.program_id(0)

    pts_t = pts_ref[...]
    ptsT = ptsT_ref[...]

    m = _dot(pts_t, ptsT)
    rA = jnp.sum(pts_t * pts_t, axis=1, keepdims=True)
    rB = jnp.sum(ptsT * ptsT, axis=0, keepdims=True)
    D = (rA - 2.0 * m) + rB

    bits = lax.bitcast_convert_type(jnp.maximum(D, 0.0), jnp.int32)
    slabs = [
        lax.bitcast_convert_type(
            lax.bitwise_or(
                lax.bitwise_and(bits[:, j * LANES:(j + 1) * LANES],
                                jnp.int32(~0xF)),
                jnp.int32(j)),
            jnp.float32)
        for j in range(nslab)
    ]

    for a, b, asc in _pruned_bitonic(nslab, keep):
        lo = jnp.minimum(slabs[a], slabs[b])
        hi = jnp.maximum(slabs[a], slabs[b])
        slabs[a], slabs[b] = (lo, hi) if asc else (hi, lo)

    stack = slabs[:keep]
    lanef = lax.broadcasted_iota(
        jnp.int32, (R, LANES), 1).astype(jnp.float32)

    for r in range(K + 1):
        heads = stack[0]
        mn = jnp.min(heads, axis=1, keepdims=True)
        cand = jnp.where(heads == mn, lanef, jnp.float32(LANES))
        lmin = jnp.min(cand, axis=1, keepdims=True)
        sel = lanef == lmin
        if r > 0:
            jr = lax.bitwise_and(
                lax.bitcast_convert_type(mn, jnp.int32), jnp.int32(0xF))
            col = lmin.astype(jnp.int32) + jr * LANES
            idx_ref[:, r - 1:r] = col + n * P
        for lv in range(keep - 1):
            stack[lv] = jnp.where(sel, stack[lv + 1], stack[lv])
        stack[keep - 1] = jnp.where(sel, jnp.float32(3.0e38),
                                    stack[keep - 1])

    g = _dot(ftile_ref[...], W0bT_ref[...])
    g_ref[...] = jnp.concatenate(
        [g, jnp.zeros((R, LANES - g.shape[1]), jnp.float32)], axis=1)


def _make_topk(N, P, R, C, Cout):
    grid = (N, P // R)
    return pl.pallas_call(
        functools.partial(_topk_body, n_pts=P, n_rows=R),
        grid=grid,
        in_specs=[
            pl.BlockSpec((None, 8, P), lambda n, t: (n, 0, 0)),
            pl.BlockSpec((None, R, 8), lambda n, t: (n, t, 0)),
            pl.BlockSpec((None, R, C), lambda n, t: (n, t, 0)),
            pl.BlockSpec((C, Cout), lambda n, t: (0, 0)),
        ],
        out_specs=[
            pl.BlockSpec((None, R, K), lambda n, t: (n, t, 0)),
            pl.BlockSpec((None, R, LANES), lambda n, t: (n, t, 0)),
        ],
        out_shape=[
            jax.ShapeDtypeStruct((N, P, K), jnp.int32),
            jax.ShapeDtypeStruct((N, P, LANES), jnp.float32),
        ],
        compiler_params=pltpu.CompilerParams(
            dimension_semantics=("parallel", "arbitrary")),
    )


_SC_BUF = 256


def _sc_gather_body(idx_hbm, table_hbm, out_hbm, idx_v, rows_v,
                    isem, gsem, osem, *, n_buf):
    wid = lax.axis_index("s") * _SC_CORES + lax.axis_index("c")
    base = wid * (n_buf * _SC_BUF)

    icps = {}
    ocps = []
    icps[0] = pltpu.make_async_copy(
        idx_hbm.at[pl.ds(base, _SC_BUF)], idx_v.at[0], isem)
    icps[0].start()
    for g in range(n_buf):
        b = g & 1
        if g >= 2:
            ocps[g - 2].wait()
        icps[g].wait()
        if g + 1 < n_buf:
            icps[g + 1] = pltpu.make_async_copy(
                idx_hbm.at[pl.ds(base + (g + 1) * _SC_BUF, _SC_BUF)],
                idx_v.at[(g + 1) & 1], isem)
            icps[g + 1].start()
        cps = []
        for j in range(_SC_BUF // _SC_CHUNK):
            cps.append(pltpu.make_async_copy(
                table_hbm.at[idx_v.at[b, pl.ds(j * _SC_CHUNK, _SC_CHUNK)]],
                rows_v.at[b, pl.ds(j * _SC_CHUNK, _SC_CHUNK)], gsem))
            cps[-1].start()
        for cp in cps:
            cp.wait()
        ocps.append(pltpu.make_async_copy(
            rows_v.at[b], out_hbm.at[pl.ds(base + g * _SC_BUF, _SC_BUF)],
            osem))
        ocps[-1].start()
    for oc in ocps[-2:]:
        oc.wait()


def _make_sc_gather(n_idx):
    n_buf = n_idx // (_SC_WORKERS * _SC_BUF)
    mesh = plsc.VectorSubcoreMesh(core_axis_name="c", subcore_axis_name="s")
    return functools.partial(
        pl.kernel,
        functools.partial(_sc_gather_body, n_buf=n_buf),
        mesh=mesh,
        out_type=jax.ShapeDtypeStruct((n_idx, LANES), jnp.float32),
        scratch_types=[
            pltpu.VMEM((2, _SC_BUF), jnp.int32),
            pltpu.VMEM((2, _SC_BUF, LANES), jnp.float32),
            pltpu.SemaphoreType.DMA,
            pltpu.SemaphoreType.DMA,
            pltpu.SemaphoreType.DMA,
        ],
        compiler_params=pltpu.CompilerParams(use_tc_tiling_on_sc=True),
    )()


def _mlp_body(f_ref, knn_ref, WcT_ref, W12_ref, W22_ref, WsT_ref,
              b0_ref, b12_ref, b22_ref, bs_ref, out_ref, *, n_rows):
    R = n_rows
    Kh = K // 2
    f_t = f_ref[...]
    Cout = out_ref.shape[-1]

    Rh = R // 2
    M = Rh * K
    h = _dot(f_t, WcT_ref[...], _FAST) + b0_ref[...]
    h_lo = jnp.broadcast_to(
        h[:Rh, None, :], (Rh, K, Cout)).reshape(M, Cout)
    h_hi = jnp.broadcast_to(
        h[Rh:, None, :], (Rh, K, Cout)).reshape(M, Cout)
    h_e = jnp.concatenate([h_lo, h_hi], axis=1)
    knn = knn_ref[...]
    paired = knn[:M, :] + pltpu.roll(knn[M:, :], 64, 1)
    e = jnp.maximum(h_e + paired, 0.0)
    e = jnp.maximum(_dot(e, W12_ref[...], _FAST) + b12_ref[...], 0.0)
    e = jnp.maximum(_dot(e, W22_ref[...], _FAST) + b22_ref[...], 0.0)
    acc = jnp.sum(e.reshape(Rh, K, 2 * Cout), axis=1)
    s = jnp.concatenate([acc[:, :Cout], acc[:, Cout:]], axis=0) * (1.0 / K)
    sc = jnp.maximum(_dot(f_t, WsT_ref[...], _FAST) + bs_ref[...], 0.0)
    out_ref[...] = s + sc


def _make_mlp(NP, R, C, Cout):
    grid = (NP // R,)
    wmap = lambda i: (0, 0)
    return pl.pallas_call(
        functools.partial(_mlp_body, n_rows=R),
        grid=grid,
        in_specs=[
            pl.BlockSpec((R, C), lambda i: (i, 0)),
            pl.BlockSpec((R * K, LANES), lambda i: (i, 0)),
            pl.BlockSpec((C, Cout), wmap),
            pl.BlockSpec((2 * Cout, 2 * Cout), wmap),
            pl.BlockSpec((2 * Cout, 2 * Cout), wmap),
            pl.BlockSpec((C, Cout), wmap),
            pl.BlockSpec((1, Cout), wmap),
            pl.BlockSpec((1, 2 * Cout), wmap),
            pl.BlockSpec((1, 2 * Cout), wmap),
            pl.BlockSpec((1, Cout), wmap),
        ],
        out_specs=pl.BlockSpec((R, Cout), lambda i: (i, 0)),
        out_shape=jax.ShapeDtypeStruct((NP, Cout), jnp.float32),
        compiler_params=pltpu.CompilerParams(
            dimension_semantics=("arbitrary",)),
    )


@jax.jit
def kernel(points, features, W0, b0, W1, b1, W2, b2, Ws, bs):
    N, P, Cpts = points.shape
    C = features.shape[-1]
    Cout = W0.shape[0]
    R = min(512, P)

    pts = jnp.concatenate(
        [points, jnp.zeros((N, P, 8 - Cpts), jnp.float32)], axis=-1)
    ptsT = jnp.transpose(pts, (0, 2, 1))

    W0a = W0[:, :C]
    W0b = W0[:, C:]
    WcT = jnp.transpose(W0a - W0b)
    W0bT = jnp.transpose(W0b)
    W1T = jnp.transpose(W1)
    W2T = jnp.transpose(W2)
    WsT = jnp.transpose(Ws)
    b0r = b0.reshape(1, Cout)
    b1r = b1.reshape(1, Cout)
    b2r = b2.reshape(1, Cout)
    bsr = bs.reshape(1, Cout)

    z = jnp.zeros((Cout, Cout), jnp.float32)
    W12 = jnp.concatenate(
        [jnp.concatenate([W1T, z], axis=1),
         jnp.concatenate([z, W1T], axis=1)], axis=0)
    W22 = jnp.concatenate(
        [jnp.concatenate([W2T, z], axis=1),
         jnp.concatenate([z, W2T], axis=1)], axis=0)
    b12 = jnp.concatenate([b1r, b1r], axis=1)
    b22 = jnp.concatenate([b2r, b2r], axis=1)

    n_half = 2 if N % 2 == 0 else 1
    Nh = N // n_half
    outs = []
    for hb in range(n_half):
        sl = slice(hb * Nh, (hb + 1) * Nh)
        f_h = features[sl]
        idx_h, g_h = _make_topk(Nh, P, R, C, Cout)(
            ptsT[sl], pts[sl], f_h, W0bT)
        n_idx = Nh * P * K
        knn_h = _make_sc_gather(n_idx)(
            idx_h.reshape(n_idx), g_h.reshape(Nh * P, LANES))
        Rc = min(512, Nh * P)
        out_h = _make_mlp(Nh * P, Rc, C, Cout)(
            f_h.reshape(Nh * P, C), knn_h, WcT, W12, W22, WsT,
            b0r, b12, b22, bsr)
        outs.append(out_h.reshape(Nh, P, Cout))
    return jnp.concatenate(outs, axis=0) if n_half > 1 else outs[0]


# speedup vs baseline: 1.0321x; 1.0321x over previous
import functools

import jax
import jax.numpy as jnp
from jax import lax
from jax.experimental import pallas as pl
from jax.experimental.pallas import tpu as pltpu
from jax.experimental.pallas import tpu_sc as plsc

K = 16
NSLAB = 16
LANES = 128
_HIGH = lax.Precision.HIGHEST
_FAST = lax.Precision.DEFAULT
_INTMAX = 0x7FFFFFFF
_KEEP = 4

_SC_CORES = 2
_SC_SUBCORES = 16
_SC_WORKERS = _SC_CORES * _SC_SUBCORES
_SC_CHUNK = 128


def _dot(a, b, precision=_HIGH):
    return lax.dot_general(a, b, (((1,), (0,)), ((), ())),
                           precision=precision,
                           preferred_element_type=jnp.float32)


def _bitonic_ce_pairs(n):
    pairs = []
    k = 2
    while k <= n:
        j = k // 2
        while j >= 1:
            for i in range(n):
                l = i ^ j
                if l > i:
                    pairs.append((i, l, (i & k) == 0))
            j //= 2
        k *= 2
    return pairs


def _pruned_bitonic(n, n_outputs):
    pairs = _bitonic_ce_pairs(n)
    needed = set(range(n_outputs))
    flags = []
    for a, b, asc in reversed(pairs):
        use = a in needed or b in needed
        flags.append(use)
        if use:
            needed.add(a)
            needed.add(b)
    flags.reverse()
    return [p for p, f in zip(pairs, flags) if f]


def _topk_body(ptsT_ref, pts_ref, ftile_ref, W0bT_ref, idx_ref, g_ref,
               *, n_pts, n_rows):
    P = n_pts
    R = n_rows
    nslab = P // LANES
    keep = min(_KEEP, nslab)
    n = pl.program_id(0)

    pts_t = pts_ref[...]
    ptsT = ptsT_ref[...]

    m = _dot(pts_t, ptsT, _FAST)
    rA = jnp.sum(pts_t * pts_t, axis=1, keepdims=True)
    rB = jnp.sum(ptsT * ptsT, axis=0, keepdims=True)
    D = (rA - 2.0 * m) + rB

    bits = lax.bitcast_convert_type(jnp.maximum(D, 0.0), jnp.int32)
    slabs = [
        lax.bitcast_convert_type(
            lax.bitwise_or(
                lax.bitwise_and(bits[:, j * LANES:(j + 1) * LANES],
                                jnp.int32(~0xF)),
                jnp.int32(j)),
            jnp.float32)
        for j in range(nslab)
    ]

    for a, b, asc in _pruned_bitonic(nslab, keep):
        lo = jnp.minimum(slabs[a], slabs[b])
        hi = jnp.maximum(slabs[a], slabs[b])
        slabs[a], slabs[b] = (lo, hi) if asc else (hi, lo)

    stack = slabs[:keep]
    lanef = lax.broadcasted_iota(
        jnp.int32, (R, LANES), 1).astype(jnp.float32)

    for r in range(K + 1):
        heads = stack[0]
        mn = jnp.min(heads, axis=1, keepdims=True)
        cand = jnp.where(heads == mn, lanef, jnp.float32(LANES))
        lmin = jnp.min(cand, axis=1, keepdims=True)
        sel = lanef == lmin
        if r > 0:
            jr = lax.bitwise_and(
                lax.bitcast_convert_type(mn, jnp.int32), jnp.int32(0xF))
            col = lmin.astype(jnp.int32) + jr * LANES
            idx_ref[:, r - 1:r] = col + n * P
        for lv in range(keep - 1):
            stack[lv] = jnp.where(sel, stack[lv + 1], stack[lv])
        stack[keep - 1] = jnp.where(sel, jnp.float32(3.0e38),
                                    stack[keep - 1])

    g = _dot(ftile_ref[...], W0bT_ref[...])
    g_ref[...] = jnp.concatenate(
        [g, jnp.zeros((R, LANES - g.shape[1]), jnp.float32)], axis=1)


def _make_topk(N, P, R, C, Cout):
    grid = (N, P // R)
    return pl.pallas_call(
        functools.partial(_topk_body, n_pts=P, n_rows=R),
        grid=grid,
        in_specs=[
            pl.BlockSpec((None, 8, P), lambda n, t: (n, 0, 0)),
            pl.BlockSpec((None, R, 8), lambda n, t: (n, t, 0)),
            pl.BlockSpec((None, R, C), lambda n, t: (n, t, 0)),
            pl.BlockSpec((C, Cout), lambda n, t: (0, 0)),
        ],
        out_specs=[
            pl.BlockSpec((None, R, K), lambda n, t: (n, t, 0)),
            pl.BlockSpec((None, R, LANES), lambda n, t: (n, t, 0)),
        ],
        out_shape=[
            jax.ShapeDtypeStruct((N, P, K), jnp.int32),
            jax.ShapeDtypeStruct((N, P, LANES), jnp.float32),
        ],
        compiler_params=pltpu.CompilerParams(
            dimension_semantics=("parallel", "arbitrary")),
    )


_SC_BUF = 256


def _sc_gather_body(idx_hbm, table_hbm, out_hbm, idx_v, rows_v,
                    isem, gsem, osem, *, n_buf):
    wid = lax.axis_index("s") * _SC_CORES + lax.axis_index("c")
    base = wid * (n_buf * _SC_BUF)

    icps = {}
    ocps = []
    icps[0] = pltpu.make_async_copy(
        idx_hbm.at[pl.ds(base, _SC_BUF)], idx_v.at[0], isem)
    icps[0].start()
    for g in range(n_buf):
        b = g & 1
        if g >= 2:
            ocps[g - 2].wait()
        icps[g].wait()
        if g + 1 < n_buf:
            icps[g + 1] = pltpu.make_async_copy(
                idx_hbm.at[pl.ds(base + (g + 1) * _SC_BUF, _SC_BUF)],
                idx_v.at[(g + 1) & 1], isem)
            icps[g + 1].start()
        cps = []
        for j in range(_SC_BUF // _SC_CHUNK):
            cps.append(pltpu.make_async_copy(
                table_hbm.at[idx_v.at[b, pl.ds(j * _SC_CHUNK, _SC_CHUNK)]],
                rows_v.at[b, pl.ds(j * _SC_CHUNK, _SC_CHUNK)], gsem))
            cps[-1].start()
        for cp in cps:
            cp.wait()
        ocps.append(pltpu.make_async_copy(
            rows_v.at[b], out_hbm.at[pl.ds(base + g * _SC_BUF, _SC_BUF)],
            osem))
        ocps[-1].start()
    for oc in ocps[-2:]:
        oc.wait()


def _make_sc_gather(n_idx):
    n_buf = n_idx // (_SC_WORKERS * _SC_BUF)
    mesh = plsc.VectorSubcoreMesh(core_axis_name="c", subcore_axis_name="s")
    return functools.partial(
        pl.kernel,
        functools.partial(_sc_gather_body, n_buf=n_buf),
        mesh=mesh,
        out_type=jax.ShapeDtypeStruct((n_idx, LANES), jnp.float32),
        scratch_types=[
            pltpu.VMEM((2, _SC_BUF), jnp.int32),
            pltpu.VMEM((2, _SC_BUF, LANES), jnp.float32),
            pltpu.SemaphoreType.DMA,
            pltpu.SemaphoreType.DMA,
            pltpu.SemaphoreType.DMA,
        ],
        compiler_params=pltpu.CompilerParams(use_tc_tiling_on_sc=True),
    )()


def _mlp_body(f_ref, knn_ref, WcT_ref, W12_ref, W22_ref, WsT_ref,
              b0_ref, b12_ref, b22_ref, bs_ref, out_ref, *, n_rows):
    R = n_rows
    Kh = K // 2
    f_t = f_ref[...]
    Cout = out_ref.shape[-1]

    Rh = R // 2
    M = Rh * K
    h = _dot(f_t, WcT_ref[...], _FAST) + b0_ref[...]
    h_lo = jnp.broadcast_to(
        h[:Rh, None, :], (Rh, K, Cout)).reshape(M, Cout)
    h_hi = jnp.broadcast_to(
        h[Rh:, None, :], (Rh, K, Cout)).reshape(M, Cout)
    h_e = jnp.concatenate([h_lo, h_hi], axis=1)
    knn = knn_ref[...]
    paired = knn[:M, :] + pltpu.roll(knn[M:, :], 64, 1)
    e = jnp.maximum(h_e + paired, 0.0)
    e = jnp.maximum(_dot(e, W12_ref[...], _FAST) + b12_ref[...], 0.0)
    e = jnp.maximum(_dot(e, W22_ref[...], _FAST) + b22_ref[...], 0.0)
    acc = jnp.sum(e.reshape(Rh, K, 2 * Cout), axis=1)
    s = jnp.concatenate([acc[:, :Cout], acc[:, Cout:]], axis=0) * (1.0 / K)
    sc = jnp.maximum(_dot(f_t, WsT_ref[...], _FAST) + bs_ref[...], 0.0)
    out_ref[...] = s + sc


def _make_mlp(NP, R, C, Cout):
    grid = (NP // R,)
    wmap = lambda i: (0, 0)
    return pl.pallas_call(
        functools.partial(_mlp_body, n_rows=R),
        grid=grid,
        in_specs=[
            pl.BlockSpec((R, C), lambda i: (i, 0)),
            pl.BlockSpec((R * K, LANES), lambda i: (i, 0)),
            pl.BlockSpec((C, Cout), wmap),
            pl.BlockSpec((2 * Cout, 2 * Cout), wmap),
            pl.BlockSpec((2 * Cout, 2 * Cout), wmap),
            pl.BlockSpec((C, Cout), wmap),
            pl.BlockSpec((1, Cout), wmap),
            pl.BlockSpec((1, 2 * Cout), wmap),
            pl.BlockSpec((1, 2 * Cout), wmap),
            pl.BlockSpec((1, Cout), wmap),
        ],
        out_specs=pl.BlockSpec((R, Cout), lambda i: (i, 0)),
        out_shape=jax.ShapeDtypeStruct((NP, Cout), jnp.float32),
        compiler_params=pltpu.CompilerParams(
            dimension_semantics=("arbitrary",)),
    )


@jax.jit
def kernel(points, features, W0, b0, W1, b1, W2, b2, Ws, bs):
    N, P, Cpts = points.shape
    C = features.shape[-1]
    Cout = W0.shape[0]
    R = min(512, P)

    pts = jnp.concatenate(
        [points, jnp.zeros((N, P, 8 - Cpts), jnp.float32)], axis=-1)
    ptsT = jnp.transpose(pts, (0, 2, 1))

    W0a = W0[:, :C]
    W0b = W0[:, C:]
    WcT = jnp.transpose(W0a - W0b)
    W0bT = jnp.transpose(W0b)
    W1T = jnp.transpose(W1)
    W2T = jnp.transpose(W2)
    WsT = jnp.transpose(Ws)
    b0r = b0.reshape(1, Cout)
    b1r = b1.reshape(1, Cout)
    b2r = b2.reshape(1, Cout)
    bsr = bs.reshape(1, Cout)

    z = jnp.zeros((Cout, Cout), jnp.float32)
    W12 = jnp.concatenate(
        [jnp.concatenate([W1T, z], axis=1),
         jnp.concatenate([z, W1T], axis=1)], axis=0)
    W22 = jnp.concatenate(
        [jnp.concatenate([W2T, z], axis=1),
         jnp.concatenate([z, W2T], axis=1)], axis=0)
    b12 = jnp.concatenate([b1r, b1r], axis=1)
    b22 = jnp.concatenate([b2r, b2r], axis=1)

    n_half = 2 if N % 2 == 0 else 1
    Nh = N // n_half
    outs = []
    for hb in range(n_half):
        sl = slice(hb * Nh, (hb + 1) * Nh)
        f_h = features[sl]
        idx_h, g_h = _make_topk(Nh, P, R, C, Cout)(
            ptsT[sl], pts[sl], f_h, W0bT)
        n_idx = Nh * P * K
        knn_h = _make_sc_gather(n_idx)(
            idx_h.reshape(n_idx), g_h.reshape(Nh * P, LANES))
        Rc = min(512, Nh * P)
        out_h = _make_mlp(Nh * P, Rc, C, Cout)(
            f_h.reshape(Nh * P, C), knn_h, WcT, W12, W22, WsT,
            b0r, b12, b22, bsr)
        outs.append(out_h.reshape(Nh, P, Cout))
    return jnp.concatenate(outs, axis=0) if n_half > 1 else outs[0]
